# Initial kernel scaffold; baseline (speedup 1.0000x reference)
import jax
import jax.numpy as jnp
from jax.experimental import pallas as pl


def kernel(center):
    raise NotImplementedError("write your pallas kernel here")


# speedup vs baseline: 39.0650x; 39.0650x over previous
import functools

import jax
import jax.numpy as jnp
from jax import lax
from jax.experimental import pallas as pl
from jax.experimental.pallas import tpu as pltpu
from jax.experimental.pallas import tpu_sc as plsc

_MASK_RATIO = 0.6
_B, _G = 256, 8192
_K = int(_MASK_RATIO * _G)
_L = 16
_CHUNKS = _G // _L
_NBINS1 = 2048
_NBINS2 = 2048
_NBINS3 = 1024


def _make_sc_kernel():
    info = plsc.get_sparse_core_info()
    nc, ns = info.num_cores, info.num_subcores
    nw = nc * ns
    rpw = _B // nw
    mesh = plsc.VectorSubcoreMesh(core_axis_name="c", subcore_axis_name="s")

    @functools.partial(
        pl.kernel,
        out_type=jax.ShapeDtypeStruct((_B, _G), jnp.float32),
        mesh=mesh,
        compiler_params=pltpu.CompilerParams(needs_layout_passes=False),
        scratch_types=[
            pltpu.VMEM((_G * 3,), jnp.float32),
            pltpu.VMEM((_G,), jnp.int32),
            pltpu.VMEM((_NBINS1,), jnp.int32),
            pltpu.VMEM((_B,), jnp.int32),
            pltpu.VMEM((_G,), jnp.float32),
        ],
    )
    def sc_mask(rows_hbm, aidx_hbm, out_hbm, rowv, bitsv, hist, aidxv, outv):
        wid = lax.axis_index("s") * nc + lax.axis_index("c")
        pltpu.sync_copy(aidx_hbm, aidxv)

        lanes = lax.broadcasted_iota(jnp.int32, (_L,), 0)
        lanes3 = lanes * 3
        ones16 = jnp.ones((_L,), jnp.int32)
        zeros16 = jnp.zeros((_L,), jnp.int32)

        def zero_hist(nbins):
            def zb(i, _):
                hist[pl.ds(i * _L, _L)] = zeros16
                return 0
            lax.fori_loop(0, nbins // _L, zb, 0)

        def scan_hist(nbins, kr):
            def sb(i, carry):
                run, bkt, pfx = carry
                v = hist[pl.ds(i * _L, _L)]
                cum = plsc.cumsum(v)
                tot = jnp.max(cum)
                cond = ((run + cum) >= kr).astype(jnp.int32)
                nbef = 16 - jnp.sum(cond)
                isnew = jnp.logical_and(bkt >= nbins, nbef < 16)
                excl = jnp.sum(jnp.where(lanes < nbef, v, 0))
                bkt = jnp.where(isnew, i * _L + nbef, bkt)
                pfx = jnp.where(isnew, run + excl, pfx)
                return (run + tot, bkt, pfx)
            _, bkt, pfx = lax.fori_loop(
                0, nbins // _L, sb,
                (jnp.int32(0), jnp.int32(nbins), jnp.int32(0)))
            return bkt, pfx

        def do_row(r, _):
            row = wid * rpw + r
            pltpu.sync_copy(rows_hbm.at[row], rowv)
            ai3 = plsc.load_gather(aidxv, [row + zeros16])[0] * 3
            av = plsc.load_gather(rowv, [ai3 + lax.rem(lanes, 3)])
            ax = av[0]
            ay = av[1]
            az = av[2]

            zero_hist(_NBINS1)

            def p1(i, _):
                idx = i * (3 * _L) + lanes3
                dx = plsc.load_gather(rowv, [idx]) - ax
                dy = plsc.load_gather(rowv, [idx + 1]) - ay
                dz = plsc.load_gather(rowv, [idx + 2]) - az
                d2 = dx * dx + dy * dy + dz * dz
                bits = plsc.bitcast(d2, jnp.int32)
                bitsv[pl.ds(i * _L, _L)] = bits
                plsc.addupdate_scatter(hist, [lax.shift_right_logical(bits, 21)],
                                       ones16)
                return 0
            lax.fori_loop(0, _CHUNKS, p1, 0)
            b1, pfx1 = scan_hist(_NBINS1, jnp.int32(_K))
            kr2 = jnp.int32(_K) - pfx1

            zero_hist(_NBINS2)

            def p2(i, _):
                bits = bitsv[pl.ds(i * _L, _L)]
                hit = (lax.shift_right_logical(bits, 21) == b1).astype(jnp.int32)
                b2i = lax.shift_right_logical(bits, 10) & (_NBINS2 - 1)
                plsc.addupdate_scatter(hist, [b2i], hit)
                return 0
            lax.fori_loop(0, _CHUNKS, p2, 0)
            b2, pfx2 = scan_hist(_NBINS2, kr2)
            kr3 = kr2 - pfx2
            top22 = b1 * _NBINS2 + b2

            zero_hist(_NBINS3)

            def p3(i, _):
                bits = bitsv[pl.ds(i * _L, _L)]
                hit = (lax.shift_right_logical(bits, 10) == top22).astype(jnp.int32)
                b3i = bits & (_NBINS3 - 1)
                plsc.addupdate_scatter(hist, [b3i], hit)
                return 0
            lax.fori_loop(0, _CHUNKS, p3, 0)
            b3, _pfx3 = scan_hist(_NBINS3, kr3)
            tbits = top22 * _NBINS3 + b3

            def pm(i, _):
                bits = bitsv[pl.ds(i * _L, _L)]
                outv[pl.ds(i * _L, _L)] = (bits <= tbits).astype(jnp.float32)
                return 0
            lax.fori_loop(0, _CHUNKS, pm, 0)
            pltpu.sync_copy(outv, out_hbm.at[row])
            return 0

        lax.fori_loop(0, rpw, do_row, 0)

    return sc_mask


def kernel(center):
    b, g, _ = center.shape
    idx_key = jax.random.key(42)
    rand_index = jax.random.randint(idx_key, (b,), 0, g)
    rows = jnp.reshape(center, (b, g * 3))
    out = _make_sc_kernel()(rows, rand_index.astype(jnp.int32))
    return out.astype(jnp.bool_)


# speedup vs baseline: 44.3111x; 1.1343x over previous
import functools

import jax
import jax.numpy as jnp
from jax import lax
from jax.experimental import pallas as pl
from jax.experimental.pallas import tpu as pltpu
from jax.experimental.pallas import tpu_sc as plsc

_MASK_RATIO = 0.6
_B, _G = 256, 8192
_K = int(_MASK_RATIO * _G)
_L = 16
_CHUNKS = _G // _L
_NBINS1 = 2048
_NBINS2 = 2048
_NBINS3 = 1024


def _make_sc_kernel():
    info = plsc.get_sparse_core_info()
    nc, ns = info.num_cores, info.num_subcores
    nw = nc * ns
    rpw = _B // nw
    mesh = plsc.VectorSubcoreMesh(core_axis_name="c", subcore_axis_name="s")

    @functools.partial(
        pl.kernel,
        out_type=jax.ShapeDtypeStruct((_B, _G), jnp.float32),
        mesh=mesh,
        compiler_params=pltpu.CompilerParams(needs_layout_passes=False),
        scratch_types=[
            pltpu.VMEM((_G * 3,), jnp.float32),
            pltpu.VMEM((_G * 3,), jnp.float32),
            pltpu.VMEM((_G,), jnp.int32),
            pltpu.VMEM((_NBINS1,), jnp.int32),
            pltpu.VMEM((_B,), jnp.int32),
            pltpu.VMEM((_G,), jnp.float32),
            pltpu.VMEM((_G + _L,), jnp.int32),
            pltpu.SemaphoreType.DMA,
            pltpu.SemaphoreType.DMA,
        ],
    )
    def sc_mask(rows_hbm, aidx_hbm, out_hbm, rowa, rowb, bitsv, hist, aidxv,
                outv, cmpv, sema, semb):
        wid = lax.axis_index("s") * nc + lax.axis_index("c")
        pltpu.sync_copy(aidx_hbm, aidxv)

        lanes = lax.broadcasted_iota(jnp.int32, (_L,), 0)
        lanes3 = lanes * 3
        ones16 = jnp.ones((_L,), jnp.int32)
        zeros16 = jnp.zeros((_L,), jnp.int32)
        sent16 = jnp.full((_L,), -1, jnp.int32)

        def zb(i, _):
            hist[pl.ds(i * _L, _L)] = zeros16
            return 0
        lax.fori_loop(0, _NBINS1 // _L, zb, 0)

        def scan_hist(nbins, kr):
            def sb(i, carry):
                run, bkt, pfx = carry
                v = hist[pl.ds(i * _L, _L)]
                hist[pl.ds(i * _L, _L)] = zeros16
                cum = plsc.cumsum(v)
                tot = jnp.max(cum)
                cond = ((run + cum) >= kr).astype(jnp.int32)
                nbef = 16 - jnp.sum(cond)
                isnew = jnp.logical_and(bkt >= nbins, nbef < 16)
                excl = jnp.sum(jnp.where(lanes < nbef, v, 0))
                bkt = jnp.where(isnew, i * _L + nbef, bkt)
                pfx = jnp.where(isnew, run + excl, pfx)
                return (run + tot, bkt, pfx)
            _, bkt, pfx = lax.fori_loop(
                0, nbins // _L, sb,
                (jnp.int32(0), jnp.int32(nbins), jnp.int32(0)))
            return bkt, pfx

        def fetch(r, buf, sem):
            pltpu.make_async_copy(
                rows_hbm.at[wid * rpw + r], buf, sem).start()

        def wait_fetch(r, buf, sem):
            pltpu.make_async_copy(
                rows_hbm.at[wid * rpw + r], buf, sem).wait()

        def process_row(r, rowv):
            row = wid * rpw + r
            ai3 = plsc.load_gather(aidxv, [row + zeros16])[0] * 3
            av = plsc.load_gather(rowv, [ai3 + lax.rem(lanes, 3)])
            ax = av[0]
            ay = av[1]
            az = av[2]

            def p1(i, _):
                idx = i * (3 * _L) + lanes3
                dx = plsc.load_gather(rowv, [idx]) - ax
                dy = plsc.load_gather(rowv, [idx + 1]) - ay
                dz = plsc.load_gather(rowv, [idx + 2]) - az
                d2 = dx * dx + dy * dy + dz * dz
                bits = plsc.bitcast(d2, jnp.int32)
                bitsv[pl.ds(i * _L, _L)] = bits
                plsc.addupdate_scatter(
                    hist, [lax.shift_right_logical(bits, 21)], ones16)
                return 0
            lax.fori_loop(0, _CHUNKS, p1, 0)
            b1, pfx1 = scan_hist(_NBINS1, jnp.int32(_K))
            kr2 = jnp.int32(_K) - pfx1

            def p2(i, coff):
                bits = bitsv[pl.ds(i * _L, _L)]
                hm = lax.shift_right_logical(bits, 21) == b1
                hit = hm.astype(jnp.int32)
                b2i = lax.shift_right_logical(bits, 10) & (_NBINS2 - 1)
                plsc.addupdate_scatter(hist, [b2i], hit)
                plsc.store_compressed(cmpv.at[pl.ds(coff, _L)], bits, mask=hm)
                return coff + jnp.sum(hit)
            m = lax.fori_loop(0, _CHUNKS, p2, jnp.int32(0))
            cmpv[pl.ds(m, _L)] = sent16
            b2, pfx2 = scan_hist(_NBINS2, kr2)
            kr3 = kr2 - pfx2
            top22 = b1 * _NBINS2 + b2

            def p3(i, _):
                bits = cmpv[pl.ds(i * _L, _L)]
                hit = (lax.shift_right_logical(bits, 10) == top22).astype(
                    jnp.int32)
                plsc.addupdate_scatter(hist, [bits & (_NBINS3 - 1)], hit)
                return 0
            lax.fori_loop(0, (m + _L - 1) // _L, p3, 0)
            b3, _pfx3 = scan_hist(_NBINS3, kr3)
            tbits = top22 * _NBINS3 + b3

            def pm(i, _):
                bits = bitsv[pl.ds(i * _L, _L)]
                outv[pl.ds(i * _L, _L)] = (bits <= tbits).astype(jnp.float32)
                return 0
            lax.fori_loop(0, _CHUNKS, pm, 0)
            pltpu.sync_copy(outv, out_hbm.at[row])

        fetch(0, rowa, sema)

        def do2(j, _):
            r0 = 2 * j
            wait_fetch(r0, rowa, sema)
            fetch(r0 + 1, rowb, semb)
            process_row(r0, rowa)
            wait_fetch(r0 + 1, rowb, semb)

            @pl.when(r0 + 2 < rpw)
            def _():
                fetch(r0 + 2, rowa, sema)
            process_row(r0 + 1, rowb)
            return 0

        lax.fori_loop(0, rpw // 2, do2, 0)

    return sc_mask


def kernel(center):
    b, g, _ = center.shape
    idx_key = jax.random.key(42)
    rand_index = jax.random.randint(idx_key, (b,), 0, g)
    rows = jnp.reshape(center, (b, g * 3))
    out = _make_sc_kernel()(rows, rand_index.astype(jnp.int32))
    return out.astype(jnp.bool_)
